# Initial kernel scaffold; baseline (speedup 1.0000x reference)
import jax
import jax.numpy as jnp
from jax.experimental import pallas as pl


def kernel(x, query_w, query_b, binding_keys, binding_values, res_w, res_b):
    raise NotImplementedError("write your pallas kernel here")


# speedup vs baseline: 27.0336x; 27.0336x over previous
import functools

import jax
import jax.numpy as jnp
from jax.experimental import pallas as pl

TOPK = 8


def _qres_kernel(x_ref, qw_ref, qb_ref, rw_ref, rb_ref, q_ref, r_ref):
    q = jax.lax.dot_general(
        x_ref[...], qw_ref[...], (((1,), (1,)), ((), ())),
        preferred_element_type=jnp.float32) + qb_ref[...]
    q_ref[...] = q
    r_ref[...] = jax.lax.dot_general(
        q, rw_ref[...], (((1,), (1,)), ((), ())),
        preferred_element_type=jnp.float32) + rb_ref[...]


def _topk_kernel(q_ref, keys_ref, probs_ref, idx_ref, *, n_keys):
    keys = keys_ref[...]
    inv = jax.lax.rsqrt(jnp.sum(keys * keys, axis=1, keepdims=True))
    keys_n = keys * inv
    scores = jax.lax.dot_general(
        q_ref[...], keys_n, (((1,), (1,)), ((), ())),
        preferred_element_type=jnp.float32)
    tb = scores.shape[0]
    iota = jax.lax.broadcasted_iota(jnp.int32, (tb, n_keys), 1)
    neg_inf = jnp.float32(-jnp.inf)
    vals = []
    idxs = []
    s = scores
    for _ in range(TOPK):
        m = jnp.max(s, axis=1, keepdims=True)
        hit = s == m
        ix = jnp.min(jnp.where(hit, iota, n_keys), axis=1, keepdims=True)
        vals.append(m)
        idxs.append(ix)
        s = jnp.where(iota == ix, neg_inf, s)
    v = jnp.concatenate(vals, axis=1)
    ix = jnp.concatenate(idxs, axis=1)
    e = jnp.exp(v - v[:, 0:1])
    probs_ref[...] = e / jnp.sum(e, axis=1, keepdims=True)
    idx_ref[...] = ix


def _combine_kernel(p_ref, i_ref, vals_ref, res_ref, out_ref, *, n_keys):
    p = p_ref[...]
    ii = i_ref[...]
    tb = p.shape[0]
    iota = jax.lax.broadcasted_iota(jnp.int32, (tb, n_keys), 1)
    w = jnp.where(iota == ii[:, 0:1], p[:, 0:1], jnp.float32(0.0))
    for j in range(1, TOPK):
        w = jnp.where(iota == ii[:, j:j + 1], p[:, j:j + 1], w)
    out = jax.lax.dot_general(
        w, vals_ref[...], (((1,), (0,)), ((), ())),
        preferred_element_type=jnp.float32)
    out_ref[...] = out + res_ref[...]


def kernel(x, query_w, query_b, binding_keys, binding_values, res_w, res_b):
    prefix = x.shape[:-1]
    d = x.shape[-1]
    bs = 1
    for p in prefix:
        bs *= p
    num_slots, n_keys, k_dim = binding_keys.shape
    v_dim = binding_values.shape[-1]
    xf = x.reshape(bs, d)

    ta = min(1024, bs)
    q, res = pl.pallas_call(
        _qres_kernel,
        grid=(bs // ta,),
        in_specs=[
            pl.BlockSpec((ta, d), lambda t: (t, 0)),
            pl.BlockSpec((k_dim, d), lambda t: (0, 0)),
            pl.BlockSpec((1, k_dim), lambda t: (0, 0)),
            pl.BlockSpec((v_dim, k_dim), lambda t: (0, 0)),
            pl.BlockSpec((1, v_dim), lambda t: (0, 0)),
        ],
        out_specs=[
            pl.BlockSpec((ta, k_dim), lambda t: (t, 0)),
            pl.BlockSpec((ta, v_dim), lambda t: (t, 0)),
        ],
        out_shape=[
            jax.ShapeDtypeStruct((bs, k_dim), jnp.float32),
            jax.ShapeDtypeStruct((bs, v_dim), jnp.float32),
        ],
    )(xf, query_w, query_b.reshape(1, k_dim), res_w, res_b.reshape(1, v_dim))

    tb = min(512, bs)
    probs, idx = pl.pallas_call(
        functools.partial(_topk_kernel, n_keys=n_keys),
        grid=(num_slots, bs // tb),
        in_specs=[
            pl.BlockSpec((tb, k_dim), lambda s, t: (t, 0)),
            pl.BlockSpec((None, n_keys, k_dim), lambda s, t: (s, 0, 0)),
        ],
        out_specs=[
            pl.BlockSpec((None, tb, TOPK), lambda s, t: (s, t, 0)),
            pl.BlockSpec((None, tb, TOPK), lambda s, t: (s, t, 0)),
        ],
        out_shape=[
            jax.ShapeDtypeStruct((num_slots, bs, TOPK), jnp.float32),
            jax.ShapeDtypeStruct((num_slots, bs, TOPK), jnp.int32),
        ],
    )(q, binding_keys)

    tc = min(256, bs)
    out = pl.pallas_call(
        functools.partial(_combine_kernel, n_keys=n_keys),
        grid=(num_slots, bs // tc),
        in_specs=[
            pl.BlockSpec((None, tc, TOPK), lambda s, t: (s, t, 0)),
            pl.BlockSpec((None, tc, TOPK), lambda s, t: (s, t, 0)),
            pl.BlockSpec((None, n_keys, v_dim), lambda s, t: (s, 0, 0)),
            pl.BlockSpec((tc, v_dim), lambda s, t: (t, 0)),
        ],
        out_specs=pl.BlockSpec((tc, v_dim), lambda s, t: (t, s)),
        out_shape=jax.ShapeDtypeStruct((bs, num_slots * v_dim), jnp.float32),
    )(probs, idx, binding_values, res)

    return out.reshape(prefix + (num_slots, v_dim))
